# baseline (speedup 1.0000x reference)
import functools

import jax
import jax.numpy as jnp
from jax import lax
from jax.experimental import pallas as pl
from jax.experimental.pallas import tpu as pltpu
from jax.experimental.pallas import tpu_sc as plsc

_ROWS = 8192
_COLS = 1024
_NC = 2
_NS = 16
_NW = _NC * _NS
_RPW = _ROWS // _NW
_CHUNK = 32
_NCHUNK = _RPW // _CHUNK
_NBUF = 2


def _sc_copy(table_hbm, out_hbm, buf0, buf1, load_sems, store_sems):
    bufs = (buf0, buf1)
    wid = lax.axis_index("s") * _NC + lax.axis_index("c")
    base = wid * _RPW

    def load(g):
        return pltpu.make_async_copy(
            table_hbm.at[pl.ds(base + g * _CHUNK, _CHUNK), :],
            bufs[g % _NBUF],
            load_sems.at[g % _NBUF],
        )

    def store(g):
        return pltpu.make_async_copy(
            bufs[g % _NBUF],
            out_hbm.at[pl.ds(base + g * _CHUNK, _CHUNK), :],
            store_sems.at[g % _NBUF],
        )

    load(0).start()
    for g in range(_NCHUNK):
        if g + 1 < _NCHUNK:
            if g + 1 >= _NBUF:
                store(g + 1 - _NBUF).wait()
            load(g + 1).start()
        load(g).wait()
        store(g).start()
    for g in range(_NCHUNK - _NBUF, _NCHUNK):
        store(g).wait()


def _run(wpe):
    k = pl.kernel(
        _sc_copy,
        out_type=jax.ShapeDtypeStruct((_ROWS, _COLS), jnp.float32),
        mesh=plsc.VectorSubcoreMesh(core_axis_name="c", subcore_axis_name="s"),
        scratch_types=[
            pltpu.VMEM((_CHUNK, _COLS), jnp.float32),
            pltpu.VMEM((_CHUNK, _COLS), jnp.float32),
            pltpu.SemaphoreType.DMA((_NBUF,)),
            pltpu.SemaphoreType.DMA((_NBUF,)),
        ],
    )
    return k(wpe)


def kernel(wpe):
    return _run(wpe).reshape(1, _ROWS, _COLS)


# speedup vs baseline: 1.0260x; 1.0260x over previous
import jax
import jax.numpy as jnp
from jax import lax
from jax.experimental import pallas as pl
from jax.experimental.pallas import tpu as pltpu
from jax.experimental.pallas import tpu_sc as plsc

_ROWS = 8192
_COLS = 1024
_NC = 2
_NS = 16
_NW = _NC * _NS
_RPW = _ROWS // _NW
_CHUNK = 32
_NCHUNK = _RPW // _CHUNK
_NBUF = 3


def _sc_copy(table_hbm, out_hbm, buf0, buf1, buf2, load_sems, store_sems):
    bufs = (buf0, buf1, buf2)
    wid = lax.axis_index("s") * _NC + lax.axis_index("c")
    base = wid * _RPW

    def load(g):
        return pltpu.make_async_copy(
            table_hbm.at[pl.ds(base + g * _CHUNK, _CHUNK), :],
            bufs[g % _NBUF],
            load_sems.at[g % _NBUF],
        )

    def store(g):
        return pltpu.make_async_copy(
            bufs[g % _NBUF],
            out_hbm.at[pl.ds(base + g * _CHUNK, _CHUNK), :],
            store_sems.at[g % _NBUF],
        )

    for g in range(_NBUF - 1):
        load(g).start()
    for g in range(_NCHUNK):
        if g + _NBUF - 1 < _NCHUNK:
            if g >= 1:
                store(g - 1).wait()
            load(g + _NBUF - 1).start()
        load(g).wait()
        store(g).start()
    for g in range(_NCHUNK - _NBUF, _NCHUNK):
        if g >= 0:
            store(g).wait()


def kernel(wpe):
    k = pl.kernel(
        _sc_copy,
        out_type=jax.ShapeDtypeStruct((_ROWS, _COLS), jnp.float32),
        mesh=plsc.VectorSubcoreMesh(core_axis_name="c", subcore_axis_name="s"),
        scratch_types=[
            pltpu.VMEM((_CHUNK, _COLS), jnp.float32),
            pltpu.VMEM((_CHUNK, _COLS), jnp.float32),
            pltpu.VMEM((_CHUNK, _COLS), jnp.float32),
            pltpu.SemaphoreType.DMA((_NBUF,)),
            pltpu.SemaphoreType.DMA((_NBUF,)),
        ],
    )
    return k(wpe).reshape(1, _ROWS, _COLS)


# speedup vs baseline: 1.0293x; 1.0032x over previous
import jax
import jax.numpy as jnp
from jax import lax
from jax.experimental import pallas as pl
from jax.experimental.pallas import tpu as pltpu
from jax.experimental.pallas import tpu_sc as plsc

_ROWS = 8192
_COLS = 1024
_NC = 2
_NS = 16
_NW = _NC * _NS
_RPW = _ROWS // _NW
_CHUNK = 16
_NCHUNK = _RPW // _CHUNK
_NBUF = 7


def _sc_copy(table_hbm, out_hbm, *rest):
    bufs = rest[:_NBUF]
    load_sems, store_sems = rest[_NBUF], rest[_NBUF + 1]
    wid = lax.axis_index("s") * _NC + lax.axis_index("c")
    base = wid * _RPW

    def load(g):
        return pltpu.make_async_copy(
            table_hbm.at[pl.ds(base + g * _CHUNK, _CHUNK), :],
            bufs[g % _NBUF],
            load_sems.at[g % _NBUF],
        )

    def store(g):
        return pltpu.make_async_copy(
            bufs[g % _NBUF],
            out_hbm.at[pl.ds(base + g * _CHUNK, _CHUNK), :],
            store_sems.at[g % _NBUF],
        )

    for g in range(_NBUF - 1):
        load(g).start()
    for g in range(_NCHUNK):
        if g + _NBUF - 1 < _NCHUNK:
            if g >= 1:
                store(g - 1).wait()
            load(g + _NBUF - 1).start()
        load(g).wait()
        store(g).start()
    for g in range(max(_NCHUNK - _NBUF, 0), _NCHUNK):
        store(g).wait()


def kernel(wpe):
    k = pl.kernel(
        _sc_copy,
        out_type=jax.ShapeDtypeStruct((_ROWS, _COLS), jnp.float32),
        mesh=plsc.VectorSubcoreMesh(core_axis_name="c", subcore_axis_name="s"),
        scratch_types=(
            [pltpu.VMEM((_CHUNK, _COLS), jnp.float32) for _ in range(_NBUF)]
            + [pltpu.SemaphoreType.DMA((_NBUF,)), pltpu.SemaphoreType.DMA((_NBUF,))]
        ),
    )
    return k(wpe).reshape(1, _ROWS, _COLS)
